# Initial kernel scaffold; baseline (speedup 1.0000x reference)
import jax
import jax.numpy as jnp
from jax.experimental import pallas as pl


def kernel(labels):
    raise NotImplementedError("write your pallas kernel here")


# speedup vs baseline: 5.6943x; 5.6943x over previous
import jax
import jax.numpy as jnp
from jax.experimental import pallas as pl
from jax.experimental.pallas import tpu as pltpu

_H = 512
_W = 512
_STRIP = 64


def _opening_mse_body(x_ref, out_ref):
    j = pl.program_id(1)
    n_strips = _H // _STRIP
    acc = jnp.zeros((_STRIP, _W), jnp.float32)
    for s in range(n_strips):
        r0 = s * _STRIP
        hi = min(r0 + _STRIP, _H - 1)
        lo = max(r0 - 1, 0)
        xs = x_ref[0, lo:hi + 1, :]
        if r0 == 0:
            xp = jnp.concatenate([xs[:1], xs], axis=0)
        else:
            xp = xs
        e = jnp.minimum(xp[1:], xp[:-1])
        el = jnp.concatenate([e[:, :1], e[:, :-1]], axis=1)
        e = jnp.minimum(e, el)
        if hi == _H - 1 and r0 + _STRIP > _H - 1:
            e = jnp.concatenate([e, e[-1:]], axis=0)
        d = jnp.maximum(e[:-1], e[1:])
        dr = jnp.concatenate([d[:, 1:], d[:, -1:]], axis=1)
        d = jnp.maximum(d, dr)
        diff = xp[1:_STRIP + 1] - d
        acc = acc + diff * diff

    total = jnp.sum(acc).reshape(1, 1, 1)

    @pl.when(j == 0)
    def _():
        out_ref[...] = total

    @pl.when(j != 0)
    def _():
        out_ref[...] = out_ref[...] + total


def kernel(labels):
    b, c, h, w = labels.shape
    n = b * c
    x = labels.reshape(n, h, w)
    per_core = n // 2
    partials = pl.pallas_call(
        _opening_mse_body,
        grid=(2, per_core),
        in_specs=[pl.BlockSpec((1, h, w), lambda i, j: (i * per_core + j, 0, 0))],
        out_specs=pl.BlockSpec((1, 1, 1), lambda i, j: (i, 0, 0)),
        out_shape=jax.ShapeDtypeStruct((2, 1, 1), jnp.float32),
        compiler_params=pltpu.CompilerParams(
            dimension_semantics=("parallel", "arbitrary"),
        ),
    )(x)
    return jnp.sum(partials) / (n * h * w)
